# baseline (speedup 1.0000x reference)
import functools

import numpy as np
import jax
import jax.numpy as jnp
from jax import lax
from jax.experimental import pallas as pl
from jax.experimental.pallas import tpu as pltpu

ROWS = 128
COLS = 100000
CHUNK = 2048
CP = 100352
R = 16

MIN_P = 0.05
LOG_MIN_P = np.float32(np.log(np.float32(MIN_P)))
TINY = np.float32(np.finfo(np.float32).tiny)
NEG_INF = np.float32(-np.inf)


def _threefry_bits(i_u32):
    k0 = np.uint32(0)
    k1 = np.uint32(42)
    ks = (k0, k1, np.uint32(k0 ^ k1 ^ np.uint32(0x1BD11BDA)))
    rot = ((13, 15, 26, 6), (17, 29, 16, 24))
    x0 = jnp.full_like(i_u32, ks[0])
    x1 = i_u32 + ks[1]
    for g in range(5):
        for r in rot[g % 2]:
            x0 = x0 + x1
            x1 = (x1 << np.uint32(r)) | (x1 >> np.uint32(32 - r))
            x1 = x0 ^ x1
        x0 = x0 + ks[(g + 1) % 3]
        x1 = x1 + ks[(g + 2) % 3] + np.uint32(g + 1)
    return x0 ^ x1


def _gumbel_from_bits(bits):
    fb = (bits >> np.uint32(9)) | np.uint32(0x3F800000)
    f = lax.bitcast_convert_type(fb, jnp.float32) - np.float32(1.0)
    u = jnp.maximum(f, TINY)
    return -jnp.log(-jnp.log(u))


def _sampler_body(l_ref, out_ref, scores_ref):
    i = pl.program_id(0)
    l = l_ref[...]
    thr = jnp.max(l, axis=1, keepdims=True) + LOG_MIN_P

    row0 = (i * R + lax.broadcasted_iota(jnp.int32, (R, CHUNK), 0))
    col_local = lax.broadcasted_iota(jnp.int32, (R, CHUNK), 1)

    def chunk_step(c, carry):
        start = pl.multiple_of(c * CHUNK, CHUNK)
        lc = l_ref[:, pl.ds(start, CHUNK)]
        flat = row0 * COLS + (c * CHUNK + col_local)
        bits = _threefry_bits(flat.astype(jnp.uint32))
        g = _gumbel_from_bits(bits)
        scores_ref[:, pl.ds(start, CHUNK)] = jnp.where(
            lc >= thr, lc + g, NEG_INF)
        return carry

    lax.fori_loop(0, CP // CHUNK, chunk_step, 0)

    s = scores_ref[...]
    mx = jnp.max(s, axis=1, keepdims=True)
    col = lax.broadcasted_iota(jnp.int32, (R, CP), 1)
    idx = jnp.min(jnp.where(s == mx, col, np.int32(CP)), axis=1)
    out_ref[...] = jnp.broadcast_to(idx[:, None], (R, 128))


@jax.jit
def kernel(logits):
    logits_p = jnp.pad(logits, ((0, 0), (0, CP - COLS)),
                       constant_values=NEG_INF)
    out = pl.pallas_call(
        _sampler_body,
        grid=(ROWS // R,),
        in_specs=[pl.BlockSpec((R, CP), lambda i: (i, 0))],
        out_specs=pl.BlockSpec((R, 128), lambda i: (i, 0)),
        out_shape=jax.ShapeDtypeStruct((ROWS, 128), jnp.int32),
        scratch_shapes=[pltpu.VMEM((R, CP), jnp.float32)],
    )(logits_p)
    return out[:, :1]
